# Initial kernel scaffold; baseline (speedup 1.0000x reference)
import jax
import jax.numpy as jnp
from jax.experimental import pallas as pl


def kernel(x, edge_index, Wl1, bl1, Wr1, Wl2, bl2, Wr2, Wl3, bl3, Wr3):
    raise NotImplementedError("write your pallas kernel here")


# speedup vs baseline: 4.5982x; 4.5982x over previous
import functools

import jax
import jax.numpy as jnp
from jax import lax
from jax.experimental import pallas as pl
from jax.experimental.pallas import tpu as pltpu
from jax.experimental.pallas import tpu_sc as plsc

N = 10000
E = 320000
D = 128

NC = 2
NS = 16
NW = NC * NS

GPW = 79
EPAD = NW * GPW * 128

NPAD = 10240
RPT = NPAD // NS


def _make_agg(compute_deg: bool):
  mesh = plsc.VectorSubcoreMesh(core_axis_name="c", subcore_axis_name="s",
                                num_cores=NC, num_subcores=NS)

  out_type = [jax.ShapeDtypeStruct((NPAD, D), jnp.float32),
              jax.ShapeDtypeStruct((NPAD, D), jnp.float32)]
  if compute_deg:
    out_type += [jax.ShapeDtypeStruct((NPAD,), jnp.float32),
                 jax.ShapeDtypeStruct((NPAD,), jnp.float32)]

  scratch = dict(
      idxs=pltpu.VMEM((GPW, 128), jnp.int32),
      idxd=pltpu.VMEM((GPW, 128), jnp.int32),
      rows=pltpu.VMEM((128, D), jnp.float32),
      acc=pltpu.VMEM_SHARED((NPAD, D), jnp.float32),
      sem=pltpu.SemaphoreType.DMA,
  )
  if compute_deg:
    scratch.update(
        ones=pltpu.VMEM((128,), jnp.float32),
        dacc=pltpu.VMEM_SHARED((NPAD,), jnp.float32),
    )

  def body(h_hbm, src_hbm, dst_hbm, zrows_hbm, zdeg_hbm,
           part0, part1, degp0, degp1,
           idxs, idxd, rows, acc, sem, ones=None, dacc=None):
    c = lax.axis_index("c")
    s = lax.axis_index("s")
    w = s * NC + c
    r0 = s * RPT

    pltpu.sync_copy(zrows_hbm, acc.at[pl.ds(r0, RPT)])
    if compute_deg:
      pltpu.sync_copy(zdeg_hbm, dacc.at[pl.ds(r0, RPT)])
      for i in range(128 // 16):
        ones[pl.ds(i * 16, 16)] = jnp.ones((16,), jnp.float32)
    plsc.subcore_barrier()

    pltpu.sync_copy(src_hbm.at[w], idxs)
    pltpu.sync_copy(dst_hbm.at[w], idxd)

    def group(j, carry):
      pltpu.async_copy(h_hbm.at[idxs.at[j]], rows, sem).wait()
      pltpu.sync_copy(rows, acc.at[idxd.at[j]], add=True)
      if compute_deg:
        pltpu.sync_copy(ones, dacc.at[idxd.at[j]], add=True)
      return carry

    lax.fori_loop(0, GPW, group, 0)
    plsc.subcore_barrier()

    @pl.when(c == 0)
    def _():
      pltpu.sync_copy(acc.at[pl.ds(r0, RPT)], part0.at[pl.ds(r0, RPT)])
      if compute_deg:
        pltpu.sync_copy(dacc.at[pl.ds(r0, RPT)], degp0.at[pl.ds(r0, RPT)])

    @pl.when(c == 1)
    def _():
      pltpu.sync_copy(acc.at[pl.ds(r0, RPT)], part1.at[pl.ds(r0, RPT)])
      if compute_deg:
        pltpu.sync_copy(dacc.at[pl.ds(r0, RPT)], degp1.at[pl.ds(r0, RPT)])

  if compute_deg:
    def wrapped(h, src, dst, zrows, zdeg, part0, part1, degp0, degp1,
                idxs=None, idxd=None, rows=None, acc=None, sem=None,
                ones=None, dacc=None):
      body(h, src, dst, zrows, zdeg, part0, part1, degp0, degp1,
           idxs, idxd, rows, acc, sem, ones, dacc)
  else:
    def wrapped(h, src, dst, zrows, part0, part1,
                idxs=None, idxd=None, rows=None, acc=None, sem=None):
      body(h, src, dst, zrows, None, part0, part1, None, None,
           idxs, idxd, rows, acc, sem)

  return pl.kernel(wrapped, out_type=tuple(out_type), mesh=mesh,
                   scratch_types=scratch)


_ROW_BLK = 1000


def _make_dense(relu: bool):
  def dense_body(p0_ref, p1_ref, d0_ref, d1_ref, h_ref, wl_ref, bl_ref,
                 wr_ref, o_ref):
    ssum = p0_ref[...] + p1_ref[...]
    d = d0_ref[...] + d1_ref[...]
    agg = ssum * (1.0 / jnp.maximum(d, 1.0))
    y = jnp.dot(agg, wl_ref[...], preferred_element_type=jnp.float32,
                precision=lax.Precision.HIGHEST)
    y = y + bl_ref[...]
    y = y + jnp.dot(h_ref[...], wr_ref[...], preferred_element_type=jnp.float32,
                    precision=lax.Precision.HIGHEST)
    o_ref[...] = jnp.maximum(y, 0.0) if relu else y

  return pl.pallas_call(
      dense_body,
      grid=(N // _ROW_BLK,),
      in_specs=[
          pl.BlockSpec((_ROW_BLK, D), lambda i: (i, 0)),
          pl.BlockSpec((_ROW_BLK, D), lambda i: (i, 0)),
          pl.BlockSpec((_ROW_BLK, 1), lambda i: (i, 0)),
          pl.BlockSpec((_ROW_BLK, 1), lambda i: (i, 0)),
          pl.BlockSpec((_ROW_BLK, D), lambda i: (i, 0)),
          pl.BlockSpec((D, D), lambda i: (0, 0)),
          pl.BlockSpec((1, D), lambda i: (0, 0)),
          pl.BlockSpec((D, D), lambda i: (0, 0)),
      ],
      out_specs=pl.BlockSpec((_ROW_BLK, D), lambda i: (i, 0)),
      out_shape=jax.ShapeDtypeStruct((N, D), jnp.float32),
  )


def kernel(x, edge_index, Wl1, bl1, Wr1, Wl2, bl2, Wr2, Wl3, bl3, Wr3):
  agg_with_deg = _make_agg(True)
  agg = _make_agg(False)
  dense_relu = _make_dense(True)
  dense_last = _make_dense(False)

  pad = EPAD - E
  src3 = jnp.concatenate(
      [edge_index[0].astype(jnp.int32), jnp.zeros((pad,), jnp.int32)]
  ).reshape(NW, GPW, 128)
  dst3 = jnp.concatenate(
      [edge_index[1].astype(jnp.int32), jnp.full((pad,), N, jnp.int32)]
  ).reshape(NW, GPW, 128)
  zrows = jnp.zeros((RPT, D), jnp.float32)
  zdeg = jnp.zeros((RPT,), jnp.float32)

  p0, p1, dg0, dg1 = agg_with_deg(x, src3, dst3, zrows, zdeg)
  dg0 = dg0.reshape(NPAD, 1)
  dg1 = dg1.reshape(NPAD, 1)
  h1 = dense_relu(p0, p1, dg0, dg1, x, Wl1, bl1.reshape(1, D), Wr1)
  p0, p1 = agg(h1, src3, dst3, zrows)
  h2 = dense_relu(p0, p1, dg0, dg1, h1, Wl2, bl2.reshape(1, D), Wr2)
  p0, p1 = agg(h2, src3, dst3, zrows)
  return dense_last(p0, p1, dg0, dg1, h2, Wl3, bl3.reshape(1, D), Wr3)
